# Initial kernel scaffold; baseline (speedup 1.0000x reference)
import jax
import jax.numpy as jnp
from jax.experimental import pallas as pl


def kernel(actors, lanes, a2a_attr, l2l_attr, a2l_rpe, params, a2a_edges, l2l_edges, a2l_edges):
    raise NotImplementedError("write your pallas kernel here")


# speedup vs baseline: 5.2371x; 5.2371x over previous
import functools
import jax
import jax.numpy as jnp
import numpy as np
from jax import lax
from jax.experimental import pallas as pl
from jax.experimental.pallas import tpu as pltpu
from jax.experimental.pallas import tpu_sc as plsc

D = 128
DRPE = 64
HEADS = 8
DH = D // HEADS
NA = 2000
NL = 8000
N = NA + NL
EA2A = 40000
EL2L = 200000
EA2L = 80000
E = EA2A + EL2L + EA2L


def _ln(x, g, b, eps=1e-5):
    m = jnp.mean(x, axis=-1, keepdims=True)
    v = jnp.mean((x - m) ** 2, axis=-1, keepdims=True)
    return g * (x - m) * jax.lax.rsqrt(v + eps) + b


_RPE_B = 400


def _rpe_body(rpe, W, b, g, beta, out):
    h = rpe[...] @ W[...] + b[...]
    out[...] = jnp.maximum(_ln(h, g[...], beta[...]), 0.0)


def _rpe_project(a2l_rpe, W, b, g, beta):
    grid = EA2L // _RPE_B
    return pl.pallas_call(
        _rpe_body,
        grid=(grid,),
        in_specs=[
            pl.BlockSpec((_RPE_B, DRPE), lambda i: (i, 0)),
            pl.BlockSpec((DRPE, D), lambda i: (0, 0)),
            pl.BlockSpec((1, D), lambda i: (0, 0)),
            pl.BlockSpec((1, D), lambda i: (0, 0)),
            pl.BlockSpec((1, D), lambda i: (0, 0)),
        ],
        out_specs=pl.BlockSpec((_RPE_B, D), lambda i: (i, 0)),
        out_shape=jax.ShapeDtypeStruct((EA2L, D), jnp.float32),
    )(a2l_rpe, W, b, g, beta)


_GW = 128
_NGRP = 2 * E // _GW
_NW = 32


def _gather_body(token_hbm, idx_hbm, out_hbm, idx_v, rows_v, sem):
    c = lax.axis_index("c")
    s = lax.axis_index("s")
    wid = s * 2 + c
    nj = (_NGRP + _NW - 1) // _NW

    def body(j, carry):
        g = wid + _NW * j

        @pl.when(g < _NGRP)
        def _():
            off = g * _GW
            pltpu.sync_copy(idx_hbm.at[pl.ds(off, _GW)], idx_v)
            pltpu.async_copy(token_hbm.at[idx_v], rows_v, sem).wait()
            pltpu.sync_copy(rows_v, out_hbm.at[pl.ds(off, _GW), :])

        return carry

    lax.fori_loop(0, nj, body, 0)


def _sc_gather(token, idx):
    mesh = plsc.VectorSubcoreMesh(core_axis_name="c", subcore_axis_name="s")
    f = pl.kernel(
        _gather_body,
        out_type=jax.ShapeDtypeStruct((2 * E, D), jnp.float32),
        mesh=mesh,
        scratch_types=[
            pltpu.VMEM((_GW,), jnp.int32),
            pltpu.VMEM((_GW, D), jnp.float32),
            pltpu.SemaphoreType.DMA,
        ],
    )
    return f(token, idx)


_EB = 512


def _edge_body(xd, xs, ea,
               w0a, w1a, w2a, mb0, mg0, mbe0,
               euw, eub, eug, eube, eng, enb,
               w0b, w1b, w2b, mb1, mg1, mbe1,
               wq, wk, wv_w, hsum,
               wv_out, ex_out):
    xd_ = xd[...]
    xs_ = xs[...]
    ea_ = ea[...]
    h = xd_ @ w0a[...] + xs_ @ w1a[...] + ea_ @ w2a[...] + mb0[...]
    mem0 = jnp.maximum(_ln(h, mg0[...], mbe0[...]), 0.0)
    delta = jnp.maximum(_ln(mem0 @ euw[...] + eub[...], eug[...], eube[...]), 0.0)
    ea1 = _ln(ea_ + delta, eng[...], enb[...])
    h1 = xd_ @ w0b[...] + xs_ @ w1b[...] + ea1 @ w2b[...] + mb1[...]
    mem1 = jnp.maximum(_ln(h1, mg1[...], mbe1[...]), 0.0)
    q = xd_ @ wq[...]
    k = mem1 @ wk[...]
    v = mem1 @ wv_w[...]
    qk = q * k
    logits = (qk @ hsum[...]) * (1.0 / np.sqrt(DH))
    ex = jnp.exp(jnp.clip(logits, -60.0, 60.0))
    wv_out[...] = ex * v
    ex_out[...] = ex


def _edge_pass(xd, xs, ea, consts):
    grid = E // _EB
    eb = pl.BlockSpec((_EB, D), lambda i: (i, 0))
    wspec = pl.BlockSpec((D, D), lambda i: (0, 0))
    vspec = pl.BlockSpec((1, D), lambda i: (0, 0))
    return pl.pallas_call(
        _edge_body,
        grid=(grid,),
        in_specs=[eb, eb, eb] + [wspec, wspec, wspec, vspec, vspec, vspec]
        + [wspec, vspec, vspec, vspec, vspec, vspec]
        + [wspec, wspec, wspec, vspec, vspec, vspec]
        + [wspec, wspec, wspec, wspec],
        out_specs=[
            pl.BlockSpec((_EB, D), lambda i: (i, 0)),
            pl.BlockSpec((_EB, D), lambda i: (i, 0)),
        ],
        out_shape=[
            jax.ShapeDtypeStruct((E, D), jnp.float32),
            jax.ShapeDtypeStruct((E, D), jnp.float32),
        ],
    )(xd, xs, ea, *consts)


_SB = 128
_EPC = E // 2
_GPC = _EPC // _SB


_NCH = 80
_NNCH = N // _NCH


def _make_scatter_body(width):
    def body_fn(val_hbm, dst_hbm, out_hbm, acc_sh, big_v, idx_v, val_v):
        c = lax.axis_index("c")
        s = lax.axis_index("s")

        def zrow(i, carry):
            for k in range(width // 16):
                big_v[i, pl.ds(k * 16, 16)] = jnp.zeros((16,), jnp.float32)
            return carry

        lax.fori_loop(0, _NCH, zrow, 0)
        ni = (_NNCH + 15) // 16

        def init(j, carry):
            ck = s + 16 * j

            @pl.when(ck < _NNCH)
            def _():
                pltpu.sync_copy(big_v, acc_sh.at[pl.ds(ck * _NCH, _NCH), :])

            return carry

        lax.fori_loop(0, ni, init, 0)
        plsc.subcore_barrier()

        nj = (_GPC + 15) // 16

        def body(j, carry):
            g = s + 16 * j

            @pl.when(g < _GPC)
            def _():
                off = c * _EPC + g * _SB
                pltpu.sync_copy(dst_hbm.at[pl.ds(off, _SB)], idx_v)
                pltpu.sync_copy(val_hbm.at[pl.ds(off, _SB), :], val_v)
                pltpu.sync_copy(val_v, acc_sh.at[idx_v], add=True)

            return carry

        lax.fori_loop(0, nj, body, 0)
        plsc.subcore_barrier()

        def fin(j, carry):
            ck = s + 16 * j

            @pl.when(ck < _NNCH)
            def _():
                row0 = ck * _NCH
                pltpu.sync_copy(acc_sh.at[pl.ds(row0, _NCH), :], big_v)
                pltpu.sync_copy(big_v, out_hbm.at[c, pl.ds(row0, _NCH), :])

            return carry

        lax.fori_loop(0, ni, fin, 0)

    return body_fn


def _sc_scatter_one(val, dst, width):
    mesh = plsc.VectorSubcoreMesh(core_axis_name="c", subcore_axis_name="s")
    f = pl.kernel(
        _make_scatter_body(width),
        out_type=jax.ShapeDtypeStruct((2, N, width), jnp.float32),
        mesh=mesh,
        scratch_types=[
            pltpu.VMEM_SHARED((N, width), jnp.float32),
            pltpu.VMEM((_NCH, width), jnp.float32),
            pltpu.VMEM((_SB,), jnp.int32),
            pltpu.VMEM((_SB, width), jnp.float32),
        ],
    )
    return f(val, dst)


def _sc_scatter(wv, ex, dst):
    return _sc_scatter_one(wv, dst, D), _sc_scatter_one(ex, dst, D)


_NB = 1000


def _node_body(tok, s2, den2, wo, n1g, n1b, f1w, f1b, f2w, f2b, n2g, n2b,
               out):
    S = s2[0] + s2[1]
    den = den2[0] + den2[1]
    recip = jnp.where(den > 0.0, 1.0 / den, 0.0)
    aggr = (S * recip) @ wo[...]
    x = _ln(tok[...] + aggr, n1g[...], n1b[...])
    ffn = jnp.maximum(x @ f1w[...] + f1b[...], 0.0) @ f2w[...] + f2b[...]
    out[...] = _ln(x + ffn, n2g[...], n2b[...])


def _node_pass(tok, s2, den2, consts):
    grid = N // _NB
    vspec = pl.BlockSpec((1, D), lambda i: (0, 0))
    wspec = pl.BlockSpec((D, D), lambda i: (0, 0))
    return pl.pallas_call(
        _node_body,
        grid=(grid,),
        in_specs=[
            pl.BlockSpec((_NB, D), lambda i: (i, 0)),
            pl.BlockSpec((2, _NB, D), lambda i: (0, i, 0)),
            pl.BlockSpec((2, _NB, D), lambda i: (0, i, 0)),
            wspec, vspec, vspec,
            pl.BlockSpec((D, 2 * D), lambda i: (0, 0)),
            pl.BlockSpec((1, 2 * D), lambda i: (0, 0)),
            pl.BlockSpec((2 * D, D), lambda i: (0, 0)),
            vspec, vspec, vspec,
        ],
        out_specs=pl.BlockSpec((_NB, D), lambda i: (i, 0)),
        out_shape=jax.ShapeDtypeStruct((N, D), jnp.float32),
    )(tok, s2, den2, *consts)


def _row(a):
    return a.reshape(1, -1)


@jax.jit
def kernel(actors, lanes, a2a_attr, l2l_attr, a2l_rpe, params, a2a_edges,
           l2l_edges, a2l_edges):
    token = jnp.concatenate([actors, lanes], axis=0)
    pr = params["proj"]
    rpe = _rpe_project(a2l_rpe, pr["W"], _row(pr["b"]), _row(pr["g"]),
                       _row(pr["beta"]))
    ea0 = jnp.concatenate([a2a_attr, l2l_attr, rpe], axis=0)
    edge_index = jnp.concatenate([a2a_edges, l2l_edges, a2l_edges], axis=1)
    src = edge_index[0]
    dst = edge_index[1]

    xg = _sc_gather(token, jnp.concatenate([src, dst]))
    xs = xg[:E]
    xd = xg[E:]

    p0 = params["layers"][0]
    p1 = params["layers"][1]
    w0a, w1a, w2a = jnp.split(p0["mp_W"], 3, axis=0)
    w0b, w1b, w2b = jnp.split(p1["mp_W"], 3, axis=0)
    li = np.arange(D) // DH
    hsum = jnp.asarray(li[:, None] == li[None, :], jnp.float32)
    consts = [
        w0a, w1a, w2a, _row(p0["mp_b"]), _row(p0["mp_g"]), _row(p0["mp_beta"]),
        p0["eu_W"], _row(p0["eu_b"]), _row(p0["eu_g"]), _row(p0["eu_beta"]),
        _row(p0["en_g"]), _row(p0["en_b"]),
        w0b, w1b, w2b, _row(p1["mp_b"]), _row(p1["mp_g"]), _row(p1["mp_beta"]),
        p1["Wq"], p1["Wk"], p1["Wv"], hsum,
    ]
    wv, ex = _edge_pass(xd, xs, ea0, consts)

    s2, den2 = _sc_scatter(wv, ex, dst)

    node_consts = [
        p1["Wo"], _row(p1["n1_g"]), _row(p1["n1_b"]),
        p1["f1_W"], _row(p1["f1_b"]), p1["f2_W"], _row(p1["f2_b"]),
        _row(p1["n2_g"]), _row(p1["n2_b"]),
    ]
    out = _node_pass(token, s2, den2, node_consts)
    return out[:NA], out[NA:]


# speedup vs baseline: 6.5936x; 1.2590x over previous
import functools
import jax
import jax.numpy as jnp
import numpy as np
from jax import lax
from jax.experimental import pallas as pl
from jax.experimental.pallas import tpu as pltpu
from jax.experimental.pallas import tpu_sc as plsc

D = 128
DRPE = 64
HEADS = 8
DH = D // HEADS
NA = 2000
NL = 8000
N = NA + NL
EA2A = 40000
EL2L = 200000
EA2L = 80000
E = EA2A + EL2L + EA2L


def _ln(x, g, b, eps=1e-5):
    m = jnp.mean(x, axis=-1, keepdims=True)
    v = jnp.mean((x - m) ** 2, axis=-1, keepdims=True)
    return g * (x - m) * jax.lax.rsqrt(v + eps) + b


_RPE_B = 400


def _rpe_body(rpe, W, b, g, beta, out):
    h = rpe[...] @ W[...] + b[...]
    out[...] = jnp.maximum(_ln(h, g[...], beta[...]), 0.0)


def _rpe_project(a2l_rpe, W, b, g, beta):
    grid = EA2L // _RPE_B
    return pl.pallas_call(
        _rpe_body,
        grid=(grid,),
        in_specs=[
            pl.BlockSpec((_RPE_B, DRPE), lambda i: (i, 0)),
            pl.BlockSpec((DRPE, D), lambda i: (0, 0)),
            pl.BlockSpec((1, D), lambda i: (0, 0)),
            pl.BlockSpec((1, D), lambda i: (0, 0)),
            pl.BlockSpec((1, D), lambda i: (0, 0)),
        ],
        out_specs=pl.BlockSpec((_RPE_B, D), lambda i: (i, 0)),
        out_shape=jax.ShapeDtypeStruct((EA2L, D), jnp.float32),
    )(a2l_rpe, W, b, g, beta)


_GW = 128
_NGRP = 2 * E // _GW
_NW = 32


def _gather_body(token_hbm, idx_hbm, out_hbm, idx_v, rows_v, sem):
    c = lax.axis_index("c")
    s = lax.axis_index("s")
    wid = s * 2 + c
    nj = (_NGRP + _NW - 1) // _NW

    def body(j, carry):
        g = wid + _NW * j

        @pl.when(g < _NGRP)
        def _():
            off = g * _GW
            pltpu.sync_copy(idx_hbm.at[pl.ds(off, _GW)], idx_v)
            pltpu.async_copy(token_hbm.at[idx_v], rows_v, sem).wait()
            pltpu.sync_copy(rows_v, out_hbm.at[pl.ds(off, _GW), :])

        return carry

    lax.fori_loop(0, nj, body, 0)


def _sc_gather(token, idx):
    mesh = plsc.VectorSubcoreMesh(core_axis_name="c", subcore_axis_name="s")
    f = pl.kernel(
        _gather_body,
        out_type=jax.ShapeDtypeStruct((2 * E, D), jnp.float32),
        mesh=mesh,
        scratch_types=[
            pltpu.VMEM((_GW,), jnp.int32),
            pltpu.VMEM((_GW, D), jnp.float32),
            pltpu.SemaphoreType.DMA,
        ],
    )
    return f(token, idx)


_EB = 1280


def _edge_body(xd, xs, ea,
               wxd, wxs, w2a, mb0, mg0, mbe0,
               euw, eub, eug, eube, eng, enb,
               w2b, mb1, mg1, mbe1,
               wkv, hsum,
               wv_out, ex_out):
    xd_ = xd[...]
    xs_ = xs[...]
    ea_ = ea[...]
    xd3 = xd_ @ wxd[...]
    xs2 = xs_ @ wxs[...]
    h = xd3[:, :D] + xs2[:, :D] + ea_ @ w2a[...] + mb0[...]
    mem0 = jnp.maximum(_ln(h, mg0[...], mbe0[...]), 0.0)
    delta = jnp.maximum(_ln(mem0 @ euw[...] + eub[...], eug[...], eube[...]), 0.0)
    ea1 = _ln(ea_ + delta, eng[...], enb[...])
    h1 = xd3[:, D:2 * D] + xs2[:, D:] + ea1 @ w2b[...] + mb1[...]
    mem1 = jnp.maximum(_ln(h1, mg1[...], mbe1[...]), 0.0)
    kv = mem1 @ wkv[...]
    qk = xd3[:, 2 * D:] * kv[:, :D]
    logits = (qk @ hsum[...]) * (1.0 / np.sqrt(DH))
    ex = jnp.exp(jnp.clip(logits, -60.0, 60.0))
    wv_out[...] = ex * kv[:, D:]
    ex_out[...] = ex


def _edge_pass(xd, xs, ea, consts):
    grid = E // _EB
    eb = pl.BlockSpec((_EB, D), lambda i: (i, 0))
    wspec = pl.BlockSpec((D, D), lambda i: (0, 0))
    vspec = pl.BlockSpec((1, D), lambda i: (0, 0))
    return pl.pallas_call(
        _edge_body,
        grid=(grid,),
        in_specs=[eb, eb, eb]
        + [pl.BlockSpec((D, 3 * D), lambda i: (0, 0)),
           pl.BlockSpec((D, 2 * D), lambda i: (0, 0)),
           wspec, vspec, vspec, vspec]
        + [wspec, vspec, vspec, vspec, vspec, vspec]
        + [wspec, vspec, vspec, vspec]
        + [pl.BlockSpec((D, 2 * D), lambda i: (0, 0)), wspec],
        out_specs=[
            pl.BlockSpec((_EB, D), lambda i: (i, 0)),
            pl.BlockSpec((_EB, D), lambda i: (i, 0)),
        ],
        out_shape=[
            jax.ShapeDtypeStruct((E, D), jnp.float32),
            jax.ShapeDtypeStruct((E, D), jnp.float32),
        ],
    )(xd, xs, ea, *consts)


_SB = 128
_EPC = E // 2
_GPC = _EPC // _SB


_NCH = 80
_NNCH = N // _NCH


def _make_scatter_body(width):
    def body_fn(val_hbm, dst_hbm, out_hbm, acc_sh, big_v, idx_v, val_v):
        c = lax.axis_index("c")
        s = lax.axis_index("s")

        def zrow(i, carry):
            for k in range(width // 16):
                big_v[i, pl.ds(k * 16, 16)] = jnp.zeros((16,), jnp.float32)
            return carry

        lax.fori_loop(0, _NCH, zrow, 0)
        ni = (_NNCH + 15) // 16

        def init(j, carry):
            ck = s + 16 * j

            @pl.when(ck < _NNCH)
            def _():
                pltpu.sync_copy(big_v, acc_sh.at[pl.ds(ck * _NCH, _NCH), :])

            return carry

        lax.fori_loop(0, ni, init, 0)
        plsc.subcore_barrier()

        nj = (_GPC + 15) // 16

        def body(j, carry):
            g = s + 16 * j

            @pl.when(g < _GPC)
            def _():
                off = c * _EPC + g * _SB
                pltpu.sync_copy(dst_hbm.at[pl.ds(off, _SB)], idx_v)
                pltpu.sync_copy(val_hbm.at[pl.ds(off, _SB), :], val_v)
                pltpu.sync_copy(val_v, acc_sh.at[idx_v], add=True)

            return carry

        lax.fori_loop(0, nj, body, 0)
        plsc.subcore_barrier()

        def fin(j, carry):
            ck = s + 16 * j

            @pl.when(ck < _NNCH)
            def _():
                row0 = ck * _NCH
                pltpu.sync_copy(acc_sh.at[pl.ds(row0, _NCH), :], big_v)
                pltpu.sync_copy(big_v, out_hbm.at[c, pl.ds(row0, _NCH), :])

            return carry

        lax.fori_loop(0, ni, fin, 0)

    return body_fn


def _sc_scatter_one(val, dst, width):
    mesh = plsc.VectorSubcoreMesh(core_axis_name="c", subcore_axis_name="s")
    f = pl.kernel(
        _make_scatter_body(width),
        out_type=jax.ShapeDtypeStruct((2, N, width), jnp.float32),
        mesh=mesh,
        scratch_types=[
            pltpu.VMEM_SHARED((N, width), jnp.float32),
            pltpu.VMEM((_NCH, width), jnp.float32),
            pltpu.VMEM((_SB,), jnp.int32),
            pltpu.VMEM((_SB, width), jnp.float32),
        ],
    )
    return f(val, dst)


def _sc_scatter(wv, ex, dst):
    return _sc_scatter_one(wv, dst, D), _sc_scatter_one(ex, dst, D)


_NB = 1000


def _node_body(tok, s2, den2, wo, n1g, n1b, f1w, f1b, f2w, f2b, n2g, n2b,
               out):
    S = s2[0] + s2[1]
    den = den2[0] + den2[1]
    recip = jnp.where(den > 0.0, 1.0 / den, 0.0)
    aggr = (S * recip) @ wo[...]
    x = _ln(tok[...] + aggr, n1g[...], n1b[...])
    ffn = jnp.maximum(x @ f1w[...] + f1b[...], 0.0) @ f2w[...] + f2b[...]
    out[...] = _ln(x + ffn, n2g[...], n2b[...])


def _node_pass(tok, s2, den2, consts):
    grid = N // _NB
    vspec = pl.BlockSpec((1, D), lambda i: (0, 0))
    wspec = pl.BlockSpec((D, D), lambda i: (0, 0))
    return pl.pallas_call(
        _node_body,
        grid=(grid,),
        in_specs=[
            pl.BlockSpec((_NB, D), lambda i: (i, 0)),
            pl.BlockSpec((2, _NB, D), lambda i: (0, i, 0)),
            pl.BlockSpec((2, _NB, D), lambda i: (0, i, 0)),
            wspec, vspec, vspec,
            pl.BlockSpec((D, 2 * D), lambda i: (0, 0)),
            pl.BlockSpec((1, 2 * D), lambda i: (0, 0)),
            pl.BlockSpec((2 * D, D), lambda i: (0, 0)),
            vspec, vspec, vspec,
        ],
        out_specs=pl.BlockSpec((_NB, D), lambda i: (i, 0)),
        out_shape=jax.ShapeDtypeStruct((N, D), jnp.float32),
    )(tok, s2, den2, *consts)


def _row(a):
    return a.reshape(1, -1)


@jax.jit
def kernel(actors, lanes, a2a_attr, l2l_attr, a2l_rpe, params, a2a_edges,
           l2l_edges, a2l_edges):
    token = jnp.concatenate([actors, lanes], axis=0)
    pr = params["proj"]
    rpe = _rpe_project(a2l_rpe, pr["W"], _row(pr["b"]), _row(pr["g"]),
                       _row(pr["beta"]))
    ea0 = jnp.concatenate([a2a_attr, l2l_attr, rpe], axis=0)
    edge_index = jnp.concatenate([a2a_edges, l2l_edges, a2l_edges], axis=1)
    src = edge_index[0]
    dst = edge_index[1]

    xg = _sc_gather(token, jnp.concatenate([src, dst]))
    xs = xg[:E]
    xd = xg[E:]

    p0 = params["layers"][0]
    p1 = params["layers"][1]
    w0a, w1a, w2a = jnp.split(p0["mp_W"], 3, axis=0)
    w0b, w1b, w2b = jnp.split(p1["mp_W"], 3, axis=0)
    li = np.arange(D) // DH
    hsum = jnp.asarray(li[:, None] == li[None, :], jnp.float32)
    wxd = jnp.concatenate([w0a, w0b, p1["Wq"]], axis=1)
    wxs = jnp.concatenate([w1a, w1b], axis=1)
    wkv = jnp.concatenate([p1["Wk"], p1["Wv"]], axis=1)
    consts = [
        wxd, wxs, w2a, _row(p0["mp_b"]), _row(p0["mp_g"]), _row(p0["mp_beta"]),
        p0["eu_W"], _row(p0["eu_b"]), _row(p0["eu_g"]), _row(p0["eu_beta"]),
        _row(p0["en_g"]), _row(p0["en_b"]),
        w2b, _row(p1["mp_b"]), _row(p1["mp_g"]), _row(p1["mp_beta"]),
        wkv, hsum,
    ]
    wv, ex = _edge_pass(xd, xs, ea0, consts)

    s2, den2 = _sc_scatter(wv, ex, dst)

    node_consts = [
        p1["Wo"], _row(p1["n1_g"]), _row(p1["n1_b"]),
        p1["f1_W"], _row(p1["f1_b"]), p1["f2_W"], _row(p1["f2_b"]),
        _row(p1["n2_g"]), _row(p1["n2_b"]),
    ]
    out = _node_pass(token, s2, den2, node_consts)
    return out[:NA], out[NA:]


# speedup vs baseline: 7.1759x; 1.0883x over previous
import functools
import jax
import jax.numpy as jnp
import numpy as np
from jax import lax
from jax.experimental import pallas as pl
from jax.experimental.pallas import tpu as pltpu
from jax.experimental.pallas import tpu_sc as plsc

D = 128
DRPE = 64
HEADS = 8
DH = D // HEADS
NA = 2000
NL = 8000
N = NA + NL
EA2A = 40000
EL2L = 200000
EA2L = 80000
E = EA2A + EL2L + EA2L


def _ln(x, g, b, eps=1e-5):
    m = jnp.mean(x, axis=-1, keepdims=True)
    v = jnp.mean((x - m) ** 2, axis=-1, keepdims=True)
    return g * (x - m) * jax.lax.rsqrt(v + eps) + b


_RPE_B = 400


def _rpe_body(rpe, W, b, g, beta, out):
    h = rpe[...] @ W[...] + b[...]
    out[...] = jnp.maximum(_ln(h, g[...], beta[...]), 0.0)


def _rpe_project(a2l_rpe, W, b, g, beta):
    grid = EA2L // _RPE_B
    return pl.pallas_call(
        _rpe_body,
        grid=(grid,),
        in_specs=[
            pl.BlockSpec((_RPE_B, DRPE), lambda i: (i, 0)),
            pl.BlockSpec((DRPE, D), lambda i: (0, 0)),
            pl.BlockSpec((1, D), lambda i: (0, 0)),
            pl.BlockSpec((1, D), lambda i: (0, 0)),
            pl.BlockSpec((1, D), lambda i: (0, 0)),
        ],
        out_specs=pl.BlockSpec((_RPE_B, D), lambda i: (i, 0)),
        out_shape=jax.ShapeDtypeStruct((EA2L, D), jnp.float32),
    )(a2l_rpe, W, b, g, beta)


_GW = 128
_NGRP = 2 * E // _GW
_NW = 32


def _gather_body(token_hbm, idx_hbm, out_hbm, idx_v, rows_v, sem):
    c = lax.axis_index("c")
    s = lax.axis_index("s")
    wid = s * 2 + c
    nj = (_NGRP + _NW - 1) // _NW

    def start(j):
        b = j & 1
        g = wid + _NW * j

        @pl.when(g < _NGRP)
        def _():
            pltpu.sync_copy(idx_hbm.at[pl.ds(g * _GW, _GW)], idx_v.at[b])
            pltpu.async_copy(token_hbm.at[idx_v.at[b]], rows_v.at[b], sem)

    def drain(j):
        b = j & 1
        g = wid + _NW * j

        @pl.when(g < _NGRP)
        def _():
            pltpu.make_async_copy(token_hbm.at[idx_v.at[b]], rows_v.at[b],
                                  sem).wait()
            pltpu.sync_copy(rows_v.at[b], out_hbm.at[pl.ds(g * _GW, _GW), :])

    start(0)

    def body(j, carry):
        start(j + 1)
        drain(j)
        return carry

    lax.fori_loop(0, nj - 1, body, 0)
    drain(nj - 1)


def _sc_gather(token, idx):
    mesh = plsc.VectorSubcoreMesh(core_axis_name="c", subcore_axis_name="s")
    f = pl.kernel(
        _gather_body,
        out_type=jax.ShapeDtypeStruct((2 * E, D), jnp.float32),
        mesh=mesh,
        scratch_types=[
            pltpu.VMEM((2, _GW), jnp.int32),
            pltpu.VMEM((2, _GW, D), jnp.float32),
            pltpu.SemaphoreType.DMA,
        ],
    )
    return f(token, idx)


_EB = 1280


def _edge_body(xd, xs, ea,
               wxd, wxs, w2a, mb0, mg0, mbe0,
               euw, eub, eug, eube, eng, enb,
               w2b, mb1, mg1, mbe1,
               wkv, hsum,
               wv_out, ex_out):
    xd_ = xd[...]
    xs_ = xs[...]
    ea_ = ea[...]
    xd3 = xd_ @ wxd[...]
    xs2 = xs_ @ wxs[...]
    h = xd3[:, :D] + xs2[:, :D] + ea_ @ w2a[...] + mb0[...]
    mem0 = jnp.maximum(_ln(h, mg0[...], mbe0[...]), 0.0)
    delta = jnp.maximum(_ln(mem0 @ euw[...] + eub[...], eug[...], eube[...]), 0.0)
    ea1 = _ln(ea_ + delta, eng[...], enb[...])
    h1 = xd3[:, D:2 * D] + xs2[:, D:] + ea1 @ w2b[...] + mb1[...]
    mem1 = jnp.maximum(_ln(h1, mg1[...], mbe1[...]), 0.0)
    kv = mem1 @ wkv[...]
    qk = xd3[:, 2 * D:] * kv[:, :D]
    logits = (qk @ hsum[...]) * (1.0 / np.sqrt(DH))
    ex = jnp.exp(jnp.clip(logits, -60.0, 60.0))
    wv_out[...] = ex * kv[:, D:]
    ex_out[...] = ex


def _edge_pass(xd, xs, ea, consts):
    grid = E // _EB
    eb = pl.BlockSpec((_EB, D), lambda i: (i, 0))
    wspec = pl.BlockSpec((D, D), lambda i: (0, 0))
    vspec = pl.BlockSpec((1, D), lambda i: (0, 0))
    return pl.pallas_call(
        _edge_body,
        grid=(grid,),
        in_specs=[eb, eb, eb]
        + [pl.BlockSpec((D, 3 * D), lambda i: (0, 0)),
           pl.BlockSpec((D, 2 * D), lambda i: (0, 0)),
           wspec, vspec, vspec, vspec]
        + [wspec, vspec, vspec, vspec, vspec, vspec]
        + [wspec, vspec, vspec, vspec]
        + [pl.BlockSpec((D, 2 * D), lambda i: (0, 0)), wspec],
        out_specs=[
            pl.BlockSpec((_EB, D), lambda i: (i, 0)),
            pl.BlockSpec((_EB, D), lambda i: (i, 0)),
        ],
        out_shape=[
            jax.ShapeDtypeStruct((E, D), jnp.float32),
            jax.ShapeDtypeStruct((E, D), jnp.float32),
        ],
    )(xd, xs, ea, *consts)


_SB = 128
_EPC = E // 2
_GPC = _EPC // _SB


_NCH = 80
_NNCH = N // _NCH


def _make_scatter_body(width):
    def body_fn(val_hbm, dst_hbm, out_hbm, acc_sh, big_v, idx_v, val_v):
        c = lax.axis_index("c")
        s = lax.axis_index("s")

        def zrow(i, carry):
            for k in range(width // 16):
                big_v[i, pl.ds(k * 16, 16)] = jnp.zeros((16,), jnp.float32)
            return carry

        lax.fori_loop(0, _NCH, zrow, 0)
        ni = (_NNCH + 15) // 16

        def init(j, carry):
            ck = s + 16 * j

            @pl.when(ck < _NNCH)
            def _():
                pltpu.sync_copy(big_v, acc_sh.at[pl.ds(ck * _NCH, _NCH), :])

            return carry

        lax.fori_loop(0, ni, init, 0)
        plsc.subcore_barrier()

        nj = (_GPC + 15) // 16

        def body(j, carry):
            g = s + 16 * j

            @pl.when(g < _GPC)
            def _():
                off = c * _EPC + g * _SB
                pltpu.sync_copy(dst_hbm.at[pl.ds(off, _SB)], idx_v)
                pltpu.sync_copy(val_hbm.at[pl.ds(off, _SB), :], val_v)
                pltpu.sync_copy(val_v, acc_sh.at[idx_v], add=True)

            return carry

        lax.fori_loop(0, nj, body, 0)
        plsc.subcore_barrier()

        def fin(j, carry):
            ck = s + 16 * j

            @pl.when(ck < _NNCH)
            def _():
                row0 = ck * _NCH
                pltpu.sync_copy(acc_sh.at[pl.ds(row0, _NCH), :], big_v)
                pltpu.sync_copy(big_v, out_hbm.at[c, pl.ds(row0, _NCH), :])

            return carry

        lax.fori_loop(0, ni, fin, 0)

    return body_fn


def _sc_scatter_one(val, dst, width):
    mesh = plsc.VectorSubcoreMesh(core_axis_name="c", subcore_axis_name="s")
    f = pl.kernel(
        _make_scatter_body(width),
        out_type=jax.ShapeDtypeStruct((2, N, width), jnp.float32),
        mesh=mesh,
        scratch_types=[
            pltpu.VMEM_SHARED((N, width), jnp.float32),
            pltpu.VMEM((_NCH, width), jnp.float32),
            pltpu.VMEM((_SB,), jnp.int32),
            pltpu.VMEM((_SB, width), jnp.float32),
        ],
    )
    return f(val, dst)


def _sc_scatter(wv, ex, dst):
    return _sc_scatter_one(wv, dst, D), _sc_scatter_one(ex, dst, D)


_NB = 1000


def _node_body(tok, s2, den2, wo, n1g, n1b, f1w, f1b, f2w, f2b, n2g, n2b,
               out):
    S = s2[0] + s2[1]
    den = den2[0] + den2[1]
    recip = jnp.where(den > 0.0, 1.0 / den, 0.0)
    aggr = (S * recip) @ wo[...]
    x = _ln(tok[...] + aggr, n1g[...], n1b[...])
    ffn = jnp.maximum(x @ f1w[...] + f1b[...], 0.0) @ f2w[...] + f2b[...]
    out[...] = _ln(x + ffn, n2g[...], n2b[...])


def _node_pass(tok, s2, den2, consts):
    grid = N // _NB
    vspec = pl.BlockSpec((1, D), lambda i: (0, 0))
    wspec = pl.BlockSpec((D, D), lambda i: (0, 0))
    return pl.pallas_call(
        _node_body,
        grid=(grid,),
        in_specs=[
            pl.BlockSpec((_NB, D), lambda i: (i, 0)),
            pl.BlockSpec((2, _NB, D), lambda i: (0, i, 0)),
            pl.BlockSpec((2, _NB, D), lambda i: (0, i, 0)),
            wspec, vspec, vspec,
            pl.BlockSpec((D, 2 * D), lambda i: (0, 0)),
            pl.BlockSpec((1, 2 * D), lambda i: (0, 0)),
            pl.BlockSpec((2 * D, D), lambda i: (0, 0)),
            vspec, vspec, vspec,
        ],
        out_specs=pl.BlockSpec((_NB, D), lambda i: (i, 0)),
        out_shape=jax.ShapeDtypeStruct((N, D), jnp.float32),
    )(tok, s2, den2, *consts)


def _row(a):
    return a.reshape(1, -1)


@jax.jit
def kernel(actors, lanes, a2a_attr, l2l_attr, a2l_rpe, params, a2a_edges,
           l2l_edges, a2l_edges):
    token = jnp.concatenate([actors, lanes], axis=0)
    pr = params["proj"]
    rpe = _rpe_project(a2l_rpe, pr["W"], _row(pr["b"]), _row(pr["g"]),
                       _row(pr["beta"]))
    ea0 = jnp.concatenate([a2a_attr, l2l_attr, rpe], axis=0)
    edge_index = jnp.concatenate([a2a_edges, l2l_edges, a2l_edges], axis=1)
    src = edge_index[0]
    dst = edge_index[1]

    xg = _sc_gather(token, jnp.concatenate([src, dst]))
    xs = xg[:E]
    xd = xg[E:]

    p0 = params["layers"][0]
    p1 = params["layers"][1]
    w0a, w1a, w2a = jnp.split(p0["mp_W"], 3, axis=0)
    w0b, w1b, w2b = jnp.split(p1["mp_W"], 3, axis=0)
    li = np.arange(D) // DH
    hsum = jnp.asarray(li[:, None] == li[None, :], jnp.float32)
    wxd = jnp.concatenate([w0a, w0b, p1["Wq"]], axis=1)
    wxs = jnp.concatenate([w1a, w1b], axis=1)
    wkv = jnp.concatenate([p1["Wk"], p1["Wv"]], axis=1)
    consts = [
        wxd, wxs, w2a, _row(p0["mp_b"]), _row(p0["mp_g"]), _row(p0["mp_beta"]),
        p0["eu_W"], _row(p0["eu_b"]), _row(p0["eu_g"]), _row(p0["eu_beta"]),
        _row(p0["en_g"]), _row(p0["en_b"]),
        w2b, _row(p1["mp_b"]), _row(p1["mp_g"]), _row(p1["mp_beta"]),
        wkv, hsum,
    ]
    wv, ex = _edge_pass(xd, xs, ea0, consts)

    s2, den2 = _sc_scatter(wv, ex, dst)

    node_consts = [
        p1["Wo"], _row(p1["n1_g"]), _row(p1["n1_b"]),
        p1["f1_W"], _row(p1["f1_b"]), p1["f2_W"], _row(p1["f2_b"]),
        _row(p1["n2_g"]), _row(p1["n2_b"]),
    ]
    out = _node_pass(token, s2, den2, node_consts)
    return out[:NA], out[NA:]


# speedup vs baseline: 8.1847x; 1.1406x over previous
import functools
import jax
import jax.numpy as jnp
import numpy as np
from jax import lax
from jax.experimental import pallas as pl
from jax.experimental.pallas import tpu as pltpu
from jax.experimental.pallas import tpu_sc as plsc

D = 128
DRPE = 64
HEADS = 8
DH = D // HEADS
NA = 2000
NL = 8000
N = NA + NL
EA2A = 40000
EL2L = 200000
EA2L = 80000
E = EA2A + EL2L + EA2L


def _ln(x, g, b, eps=1e-5):
    m = jnp.mean(x, axis=-1, keepdims=True)
    v = jnp.mean((x - m) ** 2, axis=-1, keepdims=True)
    return g * (x - m) * jax.lax.rsqrt(v + eps) + b


_RPE_B = 400


def _rpe_body(rpe, W, b, g, beta, out):
    h = rpe[...] @ W[...] + b[...]
    out[...] = jnp.maximum(_ln(h, g[...], beta[...]), 0.0)


def _rpe_project(a2l_rpe, W, b, g, beta):
    grid = EA2L // _RPE_B
    return pl.pallas_call(
        _rpe_body,
        grid=(grid,),
        in_specs=[
            pl.BlockSpec((_RPE_B, DRPE), lambda i: (i, 0)),
            pl.BlockSpec((DRPE, D), lambda i: (0, 0)),
            pl.BlockSpec((1, D), lambda i: (0, 0)),
            pl.BlockSpec((1, D), lambda i: (0, 0)),
            pl.BlockSpec((1, D), lambda i: (0, 0)),
        ],
        out_specs=pl.BlockSpec((_RPE_B, D), lambda i: (i, 0)),
        out_shape=jax.ShapeDtypeStruct((EA2L, D), jnp.float32),
    )(a2l_rpe, W, b, g, beta)


_GW = 128
_NGRP = 2 * E // _GW
_NW = 32


def _gather_body(token_hbm, idx_hbm, out_hbm, idx_v, rows_v, sem):
    c = lax.axis_index("c")
    s = lax.axis_index("s")
    wid = s * 2 + c
    nj = (_NGRP + _NW - 1) // _NW

    def start(j):
        b = j & 1
        g = wid + _NW * j

        @pl.when(g < _NGRP)
        def _():
            pltpu.sync_copy(idx_hbm.at[pl.ds(g * _GW, _GW)], idx_v.at[b])
            pltpu.async_copy(token_hbm.at[idx_v.at[b]], rows_v.at[b], sem)

    def drain(j):
        b = j & 1
        g = wid + _NW * j

        @pl.when(g < _NGRP)
        def _():
            pltpu.make_async_copy(token_hbm.at[idx_v.at[b]], rows_v.at[b],
                                  sem).wait()
            pltpu.sync_copy(rows_v.at[b], out_hbm.at[pl.ds(g * _GW, _GW), :])

    start(0)

    def body(j, carry):
        start(j + 1)
        drain(j)
        return carry

    lax.fori_loop(0, nj - 1, body, 0)
    drain(nj - 1)


def _sc_gather(token, idx):
    mesh = plsc.VectorSubcoreMesh(core_axis_name="c", subcore_axis_name="s")
    f = pl.kernel(
        _gather_body,
        out_type=jax.ShapeDtypeStruct((2 * E, D), jnp.float32),
        mesh=mesh,
        scratch_types=[
            pltpu.VMEM((2, _GW), jnp.int32),
            pltpu.VMEM((2, _GW, D), jnp.float32),
            pltpu.SemaphoreType.DMA,
        ],
    )
    return f(token, idx)


_EB = 1280


def _edge_body(xd, xs, ea,
               wxd, wxs, w2a, mb0, mg0, mbe0,
               euw, eub, eug, eube, eng, enb,
               w2b, mb1, mg1, mbe1,
               wkv, hsum,
               wv_out, ex_out):
    xd_ = xd[...]
    xs_ = xs[...]
    ea_ = ea[...]
    xd3 = xd_ @ wxd[...]
    xs2 = xs_ @ wxs[...]
    h = xd3[:, :D] + xs2[:, :D] + ea_ @ w2a[...] + mb0[...]
    mem0 = jnp.maximum(_ln(h, mg0[...], mbe0[...]), 0.0)
    delta = jnp.maximum(_ln(mem0 @ euw[...] + eub[...], eug[...], eube[...]), 0.0)
    ea1 = _ln(ea_ + delta, eng[...], enb[...])
    h1 = xd3[:, D:2 * D] + xs2[:, D:] + ea1 @ w2b[...] + mb1[...]
    mem1 = jnp.maximum(_ln(h1, mg1[...], mbe1[...]), 0.0)
    kv = mem1 @ wkv[...]
    qk = xd3[:, 2 * D:] * kv[:, :D]
    logits = (qk @ hsum[...]) * (1.0 / np.sqrt(DH))
    ex = jnp.exp(jnp.clip(logits, -60.0, 60.0))
    wv_out[...] = ex * kv[:, D:]
    ex_out[...] = ex


def _edge_pass(xd, xs, ea, consts):
    grid = E // _EB
    eb = pl.BlockSpec((_EB, D), lambda i: (i, 0))
    wspec = pl.BlockSpec((D, D), lambda i: (0, 0))
    vspec = pl.BlockSpec((1, D), lambda i: (0, 0))
    return pl.pallas_call(
        _edge_body,
        grid=(grid,),
        in_specs=[eb, eb, eb]
        + [pl.BlockSpec((D, 3 * D), lambda i: (0, 0)),
           pl.BlockSpec((D, 2 * D), lambda i: (0, 0)),
           wspec, vspec, vspec, vspec]
        + [wspec, vspec, vspec, vspec, vspec, vspec]
        + [wspec, vspec, vspec, vspec]
        + [pl.BlockSpec((D, 2 * D), lambda i: (0, 0)), wspec],
        out_specs=[
            pl.BlockSpec((_EB, D), lambda i: (i, 0)),
            pl.BlockSpec((_EB, D), lambda i: (i, 0)),
        ],
        out_shape=[
            jax.ShapeDtypeStruct((E, D), jnp.float32),
            jax.ShapeDtypeStruct((E, D), jnp.float32),
        ],
    )(xd, xs, ea, *consts)


_SB = 128
_EPC = E // 2
_GPC = _EPC // _SB


_NCH = 80
_NNCH = N // _NCH


def _scatter_body(wv_hbm, ex_hbm, dst_hbm, s_out, den_out,
                  acc_sh, big_v, idx_v, val_v, sem_i, sem_v):
    c = lax.axis_index("c")
    s = lax.axis_index("s")
    ni = (_NNCH + 15) // 16
    nj = (_GPC + 15) // 16

    def zero_big(_):
        def zrow(i, carry):
            for k in range(D // 16):
                big_v[i, pl.ds(k * 16, 16)] = jnp.zeros((16,), jnp.float32)
            return carry

        lax.fori_loop(0, _NCH, zrow, 0)

    def init_acc(_):
        def init(j, carry):
            ck = s + 16 * j

            @pl.when(ck < _NNCH)
            def _():
                pltpu.sync_copy(big_v, acc_sh.at[pl.ds(ck * _NCH, _NCH), :])

            return carry

        lax.fori_loop(0, ni, init, 0)

    def export_acc(out_hbm):
        def fin(j, carry):
            ck = s + 16 * j

            @pl.when(ck < _NNCH)
            def _():
                row0 = ck * _NCH
                pltpu.sync_copy(acc_sh.at[pl.ds(row0, _NCH), :], big_v)
                pltpu.sync_copy(big_v, out_hbm.at[c, pl.ds(row0, _NCH), :])

            return carry

        lax.fori_loop(0, ni, fin, 0)

    def accumulate(val_hbm):
        def start(j):
            b = j & 1
            g = s + 16 * j

            @pl.when(g < _GPC)
            def _():
                off = c * _EPC + g * _SB
                pltpu.async_copy(dst_hbm.at[pl.ds(off, _SB)], idx_v.at[b],
                                 sem_i)
                pltpu.async_copy(val_hbm.at[pl.ds(off, _SB), :], val_v.at[b],
                                 sem_v)

        def drain(j):
            b = j & 1
            g = s + 16 * j

            @pl.when(g < _GPC)
            def _():
                off = c * _EPC + g * _SB
                pltpu.make_async_copy(dst_hbm.at[pl.ds(off, _SB)],
                                      idx_v.at[b], sem_i).wait()
                pltpu.make_async_copy(val_hbm.at[pl.ds(off, _SB), :],
                                      val_v.at[b], sem_v).wait()
                pltpu.sync_copy(val_v.at[b], acc_sh.at[idx_v.at[b]], add=True)

        start(0)

        def body(j, carry):
            start(j + 1)
            drain(j)
            return carry

        lax.fori_loop(0, nj - 1, body, 0)
        drain(nj - 1)

    zero_big(None)
    init_acc(None)
    plsc.subcore_barrier()
    accumulate(wv_hbm)
    plsc.subcore_barrier()
    export_acc(s_out)
    plsc.subcore_barrier()
    zero_big(None)
    init_acc(None)
    plsc.subcore_barrier()
    accumulate(ex_hbm)
    plsc.subcore_barrier()
    export_acc(den_out)


def _sc_scatter(wv, ex, dst):
    mesh = plsc.VectorSubcoreMesh(core_axis_name="c", subcore_axis_name="s")
    f = pl.kernel(
        _scatter_body,
        out_type=(
            jax.ShapeDtypeStruct((2, N, D), jnp.float32),
            jax.ShapeDtypeStruct((2, N, D), jnp.float32),
        ),
        mesh=mesh,
        scratch_types=[
            pltpu.VMEM_SHARED((N, D), jnp.float32),
            pltpu.VMEM((_NCH, D), jnp.float32),
            pltpu.VMEM((2, _SB), jnp.int32),
            pltpu.VMEM((2, _SB, D), jnp.float32),
            pltpu.SemaphoreType.DMA,
            pltpu.SemaphoreType.DMA,
        ],
    )
    return f(wv, ex, dst)


_NB = 1000


def _node_body(tok, s2, den2, wo, n1g, n1b, f1w, f1b, f2w, f2b, n2g, n2b,
               out):
    S = s2[0] + s2[1]
    den = den2[0] + den2[1]
    recip = jnp.where(den > 0.0, 1.0 / den, 0.0)
    aggr = (S * recip) @ wo[...]
    x = _ln(tok[...] + aggr, n1g[...], n1b[...])
    ffn = jnp.maximum(x @ f1w[...] + f1b[...], 0.0) @ f2w[...] + f2b[...]
    out[...] = _ln(x + ffn, n2g[...], n2b[...])


def _node_pass(tok, s2, den2, consts):
    grid = N // _NB
    vspec = pl.BlockSpec((1, D), lambda i: (0, 0))
    wspec = pl.BlockSpec((D, D), lambda i: (0, 0))
    return pl.pallas_call(
        _node_body,
        grid=(grid,),
        in_specs=[
            pl.BlockSpec((_NB, D), lambda i: (i, 0)),
            pl.BlockSpec((2, _NB, D), lambda i: (0, i, 0)),
            pl.BlockSpec((2, _NB, D), lambda i: (0, i, 0)),
            wspec, vspec, vspec,
            pl.BlockSpec((D, 2 * D), lambda i: (0, 0)),
            pl.BlockSpec((1, 2 * D), lambda i: (0, 0)),
            pl.BlockSpec((2 * D, D), lambda i: (0, 0)),
            vspec, vspec, vspec,
        ],
        out_specs=pl.BlockSpec((_NB, D), lambda i: (i, 0)),
        out_shape=jax.ShapeDtypeStruct((N, D), jnp.float32),
    )(tok, s2, den2, *consts)


def _row(a):
    return a.reshape(1, -1)


@jax.jit
def kernel(actors, lanes, a2a_attr, l2l_attr, a2l_rpe, params, a2a_edges,
           l2l_edges, a2l_edges):
    token = jnp.concatenate([actors, lanes], axis=0)
    pr = params["proj"]
    rpe = _rpe_project(a2l_rpe, pr["W"], _row(pr["b"]), _row(pr["g"]),
                       _row(pr["beta"]))
    ea0 = jnp.concatenate([a2a_attr, l2l_attr, rpe], axis=0)
    edge_index = jnp.concatenate([a2a_edges, l2l_edges, a2l_edges], axis=1)
    src = edge_index[0]
    dst = edge_index[1]

    xg = _sc_gather(token, jnp.concatenate([src, dst]))
    xs = xg[:E]
    xd = xg[E:]

    p0 = params["layers"][0]
    p1 = params["layers"][1]
    w0a, w1a, w2a = jnp.split(p0["mp_W"], 3, axis=0)
    w0b, w1b, w2b = jnp.split(p1["mp_W"], 3, axis=0)
    li = np.arange(D) // DH
    hsum = jnp.asarray(li[:, None] == li[None, :], jnp.float32)
    wxd = jnp.concatenate([w0a, w0b, p1["Wq"]], axis=1)
    wxs = jnp.concatenate([w1a, w1b], axis=1)
    wkv = jnp.concatenate([p1["Wk"], p1["Wv"]], axis=1)
    consts = [
        wxd, wxs, w2a, _row(p0["mp_b"]), _row(p0["mp_g"]), _row(p0["mp_beta"]),
        p0["eu_W"], _row(p0["eu_b"]), _row(p0["eu_g"]), _row(p0["eu_beta"]),
        _row(p0["en_g"]), _row(p0["en_b"]),
        w2b, _row(p1["mp_b"]), _row(p1["mp_g"]), _row(p1["mp_beta"]),
        wkv, hsum,
    ]
    wv, ex = _edge_pass(xd, xs, ea0, consts)

    s2, den2 = _sc_scatter(wv, ex, dst)

    node_consts = [
        p1["Wo"], _row(p1["n1_g"]), _row(p1["n1_b"]),
        p1["f1_W"], _row(p1["f1_b"]), p1["f2_W"], _row(p1["f2_b"]),
        _row(p1["n2_g"]), _row(p1["n2_b"]),
    ]
    out = _node_pass(token, s2, den2, node_consts)
    return out[:NA], out[NA:]


# speedup vs baseline: 8.8589x; 1.0824x over previous
import functools
import jax
import jax.numpy as jnp
import numpy as np
from jax import lax
from jax.experimental import pallas as pl
from jax.experimental.pallas import tpu as pltpu
from jax.experimental.pallas import tpu_sc as plsc

D = 128
DRPE = 64
HEADS = 8
DH = D // HEADS
NA = 2000
NL = 8000
N = NA + NL
EA2A = 40000
EL2L = 200000
EA2L = 80000
E = EA2A + EL2L + EA2L


def _ln(x, g, b, eps=1e-5):
    m = jnp.mean(x, axis=-1, keepdims=True)
    v = jnp.mean((x - m) ** 2, axis=-1, keepdims=True)
    return g * (x - m) * jax.lax.rsqrt(v + eps) + b


_RPE_B = 400


def _rpe_body(rpe, W, b, g, beta, out):
    h = rpe[...] @ W[...] + b[...]
    out[...] = jnp.maximum(_ln(h, g[...], beta[...]), 0.0)


def _rpe_project(a2l_rpe, W, b, g, beta):
    grid = EA2L // _RPE_B
    return pl.pallas_call(
        _rpe_body,
        grid=(grid,),
        in_specs=[
            pl.BlockSpec((_RPE_B, DRPE), lambda i: (i, 0)),
            pl.BlockSpec((DRPE, D), lambda i: (0, 0)),
            pl.BlockSpec((1, D), lambda i: (0, 0)),
            pl.BlockSpec((1, D), lambda i: (0, 0)),
            pl.BlockSpec((1, D), lambda i: (0, 0)),
        ],
        out_specs=pl.BlockSpec((_RPE_B, D), lambda i: (i, 0)),
        out_shape=jax.ShapeDtypeStruct((EA2L, D), jnp.float32),
    )(a2l_rpe, W, b, g, beta)


_GW = 128
_NGRP = 2 * E // _GW
_NW = 32


def _gather_body(token_hbm, idx_hbm, out_hbm, idx_v, rows_v, sem):
    c = lax.axis_index("c")
    s = lax.axis_index("s")
    wid = s * 2 + c
    nj = (_NGRP + _NW - 1) // _NW

    def start(j):
        b = j & 1
        g = wid + _NW * j

        @pl.when(g < _NGRP)
        def _():
            pltpu.sync_copy(idx_hbm.at[pl.ds(g * _GW, _GW)], idx_v.at[b])
            pltpu.async_copy(token_hbm.at[idx_v.at[b]], rows_v.at[b], sem)

    def drain(j):
        b = j & 1
        g = wid + _NW * j

        @pl.when(g < _NGRP)
        def _():
            pltpu.make_async_copy(token_hbm.at[idx_v.at[b]], rows_v.at[b],
                                  sem).wait()
            pltpu.sync_copy(rows_v.at[b], out_hbm.at[pl.ds(g * _GW, _GW), :])

    start(0)

    def body(j, carry):
        start(j + 1)
        drain(j)
        return carry

    lax.fori_loop(0, nj - 1, body, 0)
    drain(nj - 1)


def _sc_gather(token, idx):
    mesh = plsc.VectorSubcoreMesh(core_axis_name="c", subcore_axis_name="s")
    f = pl.kernel(
        _gather_body,
        out_type=jax.ShapeDtypeStruct((2 * E, D), jnp.float32),
        mesh=mesh,
        scratch_types=[
            pltpu.VMEM((2, _GW), jnp.int32),
            pltpu.VMEM((2, _GW, D), jnp.float32),
            pltpu.SemaphoreType.DMA,
        ],
    )
    return f(token, idx)


_EB = 2560


def _edge_body(xd, xs, ea,
               wxd, wxs, w2a, mb0, mg0, mbe0,
               euw, eub, eug, eube, eng, enb,
               w2b, mb1, mg1, mbe1,
               wkv, hsum,
               wv_out, ex_out):
    xd_ = xd[...]
    xs_ = xs[...]
    ea_ = ea[...]
    xd3 = xd_ @ wxd[...]
    xs2 = xs_ @ wxs[...]
    h = xd3[:, :D] + xs2[:, :D] + ea_ @ w2a[...] + mb0[...]
    mem0 = jnp.maximum(_ln(h, mg0[...], mbe0[...]), 0.0)
    delta = jnp.maximum(_ln(mem0 @ euw[...] + eub[...], eug[...], eube[...]), 0.0)
    ea1 = _ln(ea_ + delta, eng[...], enb[...])
    h1 = xd3[:, D:2 * D] + xs2[:, D:] + ea1 @ w2b[...] + mb1[...]
    mem1 = jnp.maximum(_ln(h1, mg1[...], mbe1[...]), 0.0)
    kv = mem1 @ wkv[...]
    qk = xd3[:, 2 * D:] * kv[:, :D]
    logits = (qk @ hsum[...]) * (1.0 / np.sqrt(DH))
    ex = jnp.exp(jnp.clip(logits, -60.0, 60.0))
    wv_out[...] = ex * kv[:, D:]
    ex_out[...] = ex


def _edge_pass(xd, xs, ea, consts):
    grid = E // _EB
    eb = pl.BlockSpec((_EB, D), lambda i: (i, 0))
    wspec = pl.BlockSpec((D, D), lambda i: (0, 0))
    vspec = pl.BlockSpec((1, D), lambda i: (0, 0))
    return pl.pallas_call(
        _edge_body,
        grid=(grid,),
        in_specs=[eb, eb, eb]
        + [pl.BlockSpec((D, 3 * D), lambda i: (0, 0)),
           pl.BlockSpec((D, 2 * D), lambda i: (0, 0)),
           wspec, vspec, vspec, vspec]
        + [wspec, vspec, vspec, vspec, vspec, vspec]
        + [wspec, vspec, vspec, vspec]
        + [pl.BlockSpec((D, 2 * D), lambda i: (0, 0)), wspec],
        out_specs=[
            pl.BlockSpec((_EB, D), lambda i: (i, 0)),
            pl.BlockSpec((_EB, D), lambda i: (i, 0)),
        ],
        out_shape=[
            jax.ShapeDtypeStruct((E, D), jnp.float32),
            jax.ShapeDtypeStruct((E, D), jnp.float32),
        ],
    )(xd, xs, ea, *consts)


_SB = 128
_EPC = E // 2
_GPC = _EPC // _SB


_NCH = 80
_NNCH = N // _NCH


def _scatter_body(wv_hbm, ex_hbm, dst_hbm, s_out, den_out,
                  acc_sh, big_v, idx_v, val_v, sem_i, sem_v):
    c = lax.axis_index("c")
    s = lax.axis_index("s")
    ni = (_NNCH + 15) // 16
    nj = (_GPC + 15) // 16

    def zero_big(_):
        def zrow(i, carry):
            for k in range(D // 16):
                big_v[i, pl.ds(k * 16, 16)] = jnp.zeros((16,), jnp.float32)
            return carry

        lax.fori_loop(0, _NCH, zrow, 0)

    def init_acc(_):
        def init(j, carry):
            ck = s + 16 * j

            @pl.when(ck < _NNCH)
            def _():
                pltpu.sync_copy(big_v, acc_sh.at[pl.ds(ck * _NCH, _NCH), :])

            return carry

        lax.fori_loop(0, ni, init, 0)

    def export_acc(out_hbm):
        def fin(j, carry):
            ck = s + 16 * j

            @pl.when(ck < _NNCH)
            def _():
                row0 = ck * _NCH
                pltpu.sync_copy(acc_sh.at[pl.ds(row0, _NCH), :], big_v)
                pltpu.sync_copy(big_v, out_hbm.at[c, pl.ds(row0, _NCH), :])

            return carry

        lax.fori_loop(0, ni, fin, 0)

    def accumulate(val_hbm):
        def start(j):
            b = j & 1
            g = s + 16 * j

            @pl.when(g < _GPC)
            def _():
                off = c * _EPC + g * _SB
                pltpu.async_copy(dst_hbm.at[pl.ds(off, _SB)], idx_v.at[b],
                                 sem_i)
                pltpu.async_copy(val_hbm.at[pl.ds(off, _SB), :], val_v.at[b],
                                 sem_v)

        def drain(j):
            b = j & 1
            g = s + 16 * j

            @pl.when(g < _GPC)
            def _():
                off = c * _EPC + g * _SB
                pltpu.make_async_copy(dst_hbm.at[pl.ds(off, _SB)],
                                      idx_v.at[b], sem_i).wait()
                pltpu.make_async_copy(val_hbm.at[pl.ds(off, _SB), :],
                                      val_v.at[b], sem_v).wait()
                pltpu.sync_copy(val_v.at[b], acc_sh.at[idx_v.at[b]], add=True)

        start(0)

        def body(j, carry):
            start(j + 1)
            drain(j)
            return carry

        lax.fori_loop(0, nj - 1, body, 0)
        drain(nj - 1)

    zero_big(None)
    init_acc(None)
    plsc.subcore_barrier()
    accumulate(wv_hbm)
    plsc.subcore_barrier()
    export_acc(s_out)
    plsc.subcore_barrier()
    zero_big(None)
    init_acc(None)
    plsc.subcore_barrier()
    accumulate(ex_hbm)
    plsc.subcore_barrier()
    export_acc(den_out)


def _sc_scatter(wv, ex, dst):
    mesh = plsc.VectorSubcoreMesh(core_axis_name="c", subcore_axis_name="s")
    f = pl.kernel(
        _scatter_body,
        out_type=(
            jax.ShapeDtypeStruct((2, N, D), jnp.float32),
            jax.ShapeDtypeStruct((2, N, D), jnp.float32),
        ),
        mesh=mesh,
        scratch_types=[
            pltpu.VMEM_SHARED((N, D), jnp.float32),
            pltpu.VMEM((_NCH, D), jnp.float32),
            pltpu.VMEM((2, _SB), jnp.int32),
            pltpu.VMEM((2, _SB, D), jnp.float32),
            pltpu.SemaphoreType.DMA,
            pltpu.SemaphoreType.DMA,
        ],
    )
    return f(wv, ex, dst)


_NB = 1000


def _node_body(tok, s2, den2, wo, n1g, n1b, f1w, f1b, f2w, f2b, n2g, n2b,
               out):
    S = s2[0] + s2[1]
    den = den2[0] + den2[1]
    recip = jnp.where(den > 0.0, 1.0 / den, 0.0)
    aggr = (S * recip) @ wo[...]
    x = _ln(tok[...] + aggr, n1g[...], n1b[...])
    ffn = jnp.maximum(x @ f1w[...] + f1b[...], 0.0) @ f2w[...] + f2b[...]
    out[...] = _ln(x + ffn, n2g[...], n2b[...])


def _node_pass(tok, s2, den2, consts):
    grid = N // _NB
    vspec = pl.BlockSpec((1, D), lambda i: (0, 0))
    wspec = pl.BlockSpec((D, D), lambda i: (0, 0))
    return pl.pallas_call(
        _node_body,
        grid=(grid,),
        in_specs=[
            pl.BlockSpec((_NB, D), lambda i: (i, 0)),
            pl.BlockSpec((2, _NB, D), lambda i: (0, i, 0)),
            pl.BlockSpec((2, _NB, D), lambda i: (0, i, 0)),
            wspec, vspec, vspec,
            pl.BlockSpec((D, 2 * D), lambda i: (0, 0)),
            pl.BlockSpec((1, 2 * D), lambda i: (0, 0)),
            pl.BlockSpec((2 * D, D), lambda i: (0, 0)),
            vspec, vspec, vspec,
        ],
        out_specs=pl.BlockSpec((_NB, D), lambda i: (i, 0)),
        out_shape=jax.ShapeDtypeStruct((N, D), jnp.float32),
    )(tok, s2, den2, *consts)


def _row(a):
    return a.reshape(1, -1)


@jax.jit
def kernel(actors, lanes, a2a_attr, l2l_attr, a2l_rpe, params, a2a_edges,
           l2l_edges, a2l_edges):
    token = jnp.concatenate([actors, lanes], axis=0)
    pr = params["proj"]
    rpe = _rpe_project(a2l_rpe, pr["W"], _row(pr["b"]), _row(pr["g"]),
                       _row(pr["beta"]))
    ea0 = jnp.concatenate([a2a_attr, l2l_attr, rpe], axis=0)
    edge_index = jnp.concatenate([a2a_edges, l2l_edges, a2l_edges], axis=1)
    src = edge_index[0]
    dst = edge_index[1]

    xg = _sc_gather(token, jnp.concatenate([src, dst]))
    xs = xg[:E]
    xd = xg[E:]

    p0 = params["layers"][0]
    p1 = params["layers"][1]
    w0a, w1a, w2a = jnp.split(p0["mp_W"], 3, axis=0)
    w0b, w1b, w2b = jnp.split(p1["mp_W"], 3, axis=0)
    li = np.arange(D) // DH
    hsum = jnp.asarray(li[:, None] == li[None, :], jnp.float32)
    wxd = jnp.concatenate([w0a, w0b, p1["Wq"]], axis=1)
    wxs = jnp.concatenate([w1a, w1b], axis=1)
    wkv = jnp.concatenate([p1["Wk"], p1["Wv"]], axis=1)
    consts = [
        wxd, wxs, w2a, _row(p0["mp_b"]), _row(p0["mp_g"]), _row(p0["mp_beta"]),
        p0["eu_W"], _row(p0["eu_b"]), _row(p0["eu_g"]), _row(p0["eu_beta"]),
        _row(p0["en_g"]), _row(p0["en_b"]),
        w2b, _row(p1["mp_b"]), _row(p1["mp_g"]), _row(p1["mp_beta"]),
        wkv, hsum,
    ]
    wv, ex = _edge_pass(xd, xs, ea0, consts)

    s2, den2 = _sc_scatter(wv, ex, dst)

    node_consts = [
        p1["Wo"], _row(p1["n1_g"]), _row(p1["n1_b"]),
        p1["f1_W"], _row(p1["f1_b"]), p1["f2_W"], _row(p1["f2_b"]),
        _row(p1["n2_g"]), _row(p1["n2_b"]),
    ]
    out = _node_pass(token, s2, den2, node_consts)
    return out[:NA], out[NA:]
